# baseline (speedup 1.0000x reference)
import functools

import jax
import jax.numpy as jnp
from jax import lax
from jax.experimental import pallas as pl
from jax.experimental.pallas import tpu as pltpu
from jax.experimental.pallas import tpu_sc as plsc

L = 16
NBUF = 3
NPAIR = 8
UNROLL = 32
SC_SEQ = 1088
TC_BS = 64


def _lane_top2_insert(m1, m2, v):
    mn = jnp.minimum(m1, v)
    return jnp.maximum(m1, v), jnp.maximum(m2, mn)


def _pair_merge(a, b):
    a1, a2 = a
    b1, b2 = b
    hi = jnp.maximum(a1, b1)
    mid = jnp.minimum(a1, b1)
    lo = jnp.maximum(jnp.maximum(a2, b2), mid)
    return hi, lo


def _make_sc_kernel(V, n_workers, seq_len, sc_seq):
    mesh = plsc.VectorSubcoreMesh(core_axis_name="c", subcore_axis_name="s")
    num_cores = mesh.num_cores

    @functools.partial(
        pl.kernel,
        out_type=jax.ShapeDtypeStruct((n_workers, L), jnp.float32),
        mesh=mesh,
        compiler_params=pltpu.CompilerParams(needs_layout_passes=False),
        scratch_types=[
            pltpu.VMEM((V,), jnp.float32),
            pltpu.VMEM((V,), jnp.float32),
            pltpu.VMEM((V,), jnp.float32),
            pltpu.VMEM((L,), jnp.float32),
            pltpu.SemaphoreType.DMA,
            pltpu.SemaphoreType.DMA,
            pltpu.SemaphoreType.DMA,
        ],
    )
    def sc_kernel(x_hbm, out_hbm, buf0, buf1, buf2, outbuf, sem0, sem1, sem2):
        bufs = (buf0, buf1, buf2)
        sems = (sem0, sem1, sem2)
        wid = lax.axis_index("s") * num_cores + lax.axis_index("c")
        base = wid * seq_len

        for b in range(NBUF):
            pltpu.make_async_copy(x_hbm.at[base + b], bufs[b], sems[b]).start()

        neg = jnp.full((L,), -3.0e38, jnp.float32)

        def row_top2(b):
            def inner(i, carry):
                pairs = [list(carry[2 * j:2 * j + 2]) for j in range(NPAIR)]
                o = i * (UNROLL * L)
                for k in range(UNROLL):
                    v = bufs[b][pl.ds(o + k * L, L)]
                    j = k % NPAIR
                    pairs[j][0], pairs[j][1] = _lane_top2_insert(
                        pairs[j][0], pairs[j][1], v)
                return tuple(x for p in pairs for x in p)

            res = lax.fori_loop(0, V // (UNROLL * L), inner, (neg,) * (2 * NPAIR))
            pairs = [(res[2 * j], res[2 * j + 1]) for j in range(NPAIR)]
            while len(pairs) > 1:
                pairs = [_pair_merge(pairs[i], pairs[i + 1])
                         for i in range(0, len(pairs), 2)]
            m1, m2 = pairs[0]

            top1 = jnp.max(m1)
            eq = m1 == top1
            firsts = jnp.cumsum(eq.astype(jnp.int32))
            first = jnp.logical_and(eq, firsts == 1)
            merged = jnp.where(first, m2, m1)
            top2 = jnp.max(merged)
            return top1 - top2

        def outer(g, acc):
            for b in range(NBUF):
                pltpu.make_async_copy(x_hbm.at[base], bufs[b], sems[b]).wait()
                gap = row_top2(b)
                acc = acc + gap
                nxt = g * NBUF + b + NBUF

                @pl.when(nxt < sc_seq)
                def _():
                    pltpu.make_async_copy(
                        x_hbm.at[base + nxt], bufs[b], sems[b]).start()
            return acc

        acc = lax.fori_loop(0, sc_seq // NBUF, outer,
                            jnp.zeros((L,), jnp.float32))
        for b in range(sc_seq % NBUF):
            pltpu.make_async_copy(x_hbm.at[base], bufs[b], sems[b]).wait()
            acc = acc + row_top2(b)
        outbuf[...] = acc
        pltpu.sync_copy(outbuf, out_hbm.at[wid])

    return sc_kernel


def _make_tc_kernel(B, S, V, s0, bs):
    nblk = (S - s0) // bs

    def tc_body(x_ref, o_ref):
        j = pl.program_id(1)
        x = x_ref[...]
        neg = jnp.float32(-3.0e38)
        m1 = jnp.max(x, axis=-1)
        eq = x == m1[..., None]
        cnt = jnp.sum(eq.astype(jnp.float32), axis=-1)
        m2 = jnp.max(jnp.where(eq, neg, x), axis=-1)
        m2 = jnp.where(cnt > 1.5, m1, m2)
        gsum = jnp.sum(m1 - m2)

        @pl.when(j == 0)
        def _():
            o_ref[...] = jnp.zeros_like(o_ref)

        o_ref[...] += jnp.full((1, 1, 128), gsum, jnp.float32)

    return pl.pallas_call(
        tc_body,
        grid=(B, nblk),
        in_specs=[pl.BlockSpec((1, bs, V), lambda b, j: (b, s0 // bs + j, 0))],
        out_specs=pl.BlockSpec((1, 1, 128), lambda b, j: (b, 0, 0)),
        out_shape=jax.ShapeDtypeStruct((B, 1, 128), jnp.float32),
    )


def kernel(logits):
    B, S, V = logits.shape
    n_workers = 2 * 16
    x = logits.reshape(B * S, V)
    sc_sum = _make_sc_kernel(V, n_workers, S, SC_SEQ)(x)[:, 0]
    if SC_SEQ < S:
        tc_sum = _make_tc_kernel(B, S, V, SC_SEQ, TC_BS)(logits)[:, 0, 0]
        gap = (sc_sum + tc_sum) * jnp.float32(1.0 / S)
    else:
        gap = sc_sum * jnp.float32(1.0 / S)
    avg_gap = jnp.mean(gap)
    should_exit = avg_gap >= jnp.float32(7.5)
    return gap, avg_gap, should_exit


# speedup vs baseline: 1.0036x; 1.0036x over previous
import functools

import jax
import jax.numpy as jnp
from jax import lax
from jax.experimental import pallas as pl
from jax.experimental.pallas import tpu as pltpu
from jax.experimental.pallas import tpu_sc as plsc

L = 16
NBUF = 3
NPAIR = 8
UNROLL = 32
SC_SEQ = 1024
TC_BS = 64


def _lane_top2_insert(m1, m2, v):
    mn = jnp.minimum(m1, v)
    return jnp.maximum(m1, v), jnp.maximum(m2, mn)


def _pair_merge(a, b):
    a1, a2 = a
    b1, b2 = b
    hi = jnp.maximum(a1, b1)
    mid = jnp.minimum(a1, b1)
    lo = jnp.maximum(jnp.maximum(a2, b2), mid)
    return hi, lo


def _make_sc_kernel(V, n_workers, seq_len, sc_seq):
    mesh = plsc.VectorSubcoreMesh(core_axis_name="c", subcore_axis_name="s")
    num_cores = mesh.num_cores

    @functools.partial(
        pl.kernel,
        out_type=jax.ShapeDtypeStruct((n_workers, L), jnp.float32),
        mesh=mesh,
        compiler_params=pltpu.CompilerParams(needs_layout_passes=False),
        scratch_types=[
            pltpu.VMEM((V,), jnp.float32),
            pltpu.VMEM((V,), jnp.float32),
            pltpu.VMEM((V,), jnp.float32),
            pltpu.VMEM((L,), jnp.float32),
            pltpu.SemaphoreType.DMA,
            pltpu.SemaphoreType.DMA,
            pltpu.SemaphoreType.DMA,
        ],
    )
    def sc_kernel(x_hbm, out_hbm, buf0, buf1, buf2, outbuf, sem0, sem1, sem2):
        bufs = (buf0, buf1, buf2)
        sems = (sem0, sem1, sem2)
        wid = lax.axis_index("s") * num_cores + lax.axis_index("c")
        base = wid * seq_len

        for b in range(NBUF):
            pltpu.make_async_copy(x_hbm.at[base + b], bufs[b], sems[b]).start()

        neg = jnp.full((L,), -3.0e38, jnp.float32)

        def row_top2(b):
            def inner(i, carry):
                pairs = [list(carry[2 * j:2 * j + 2]) for j in range(NPAIR)]
                o = i * (UNROLL * L)
                for k in range(UNROLL):
                    v = bufs[b][pl.ds(o + k * L, L)]
                    j = k % NPAIR
                    pairs[j][0], pairs[j][1] = _lane_top2_insert(
                        pairs[j][0], pairs[j][1], v)
                return tuple(x for p in pairs for x in p)

            res = lax.fori_loop(0, V // (UNROLL * L), inner, (neg,) * (2 * NPAIR))
            pairs = [(res[2 * j], res[2 * j + 1]) for j in range(NPAIR)]
            while len(pairs) > 1:
                pairs = [_pair_merge(pairs[i], pairs[i + 1])
                         for i in range(0, len(pairs), 2)]
            m1, m2 = pairs[0]

            top1 = jnp.max(m1)
            eq = m1 == top1
            firsts = jnp.cumsum(eq.astype(jnp.int32))
            first = jnp.logical_and(eq, firsts == 1)
            merged = jnp.where(first, m2, m1)
            top2 = jnp.max(merged)
            return top1 - top2

        def outer(g, acc):
            for b in range(NBUF):
                pltpu.make_async_copy(x_hbm.at[base], bufs[b], sems[b]).wait()
                gap = row_top2(b)
                acc = acc + gap
                nxt = g * NBUF + b + NBUF

                @pl.when(nxt < sc_seq)
                def _():
                    pltpu.make_async_copy(
                        x_hbm.at[base + nxt], bufs[b], sems[b]).start()
            return acc

        acc = lax.fori_loop(0, sc_seq // NBUF, outer,
                            jnp.zeros((L,), jnp.float32))
        for b in range(sc_seq % NBUF):
            pltpu.make_async_copy(x_hbm.at[base], bufs[b], sems[b]).wait()
            acc = acc + row_top2(b)
        outbuf[...] = acc
        pltpu.sync_copy(outbuf, out_hbm.at[wid])

    return sc_kernel


def _make_tc_kernel(B, S, V, s0, bs):
    nblk = (S - s0) // bs

    def tc_body(x_ref, o_ref):
        j = pl.program_id(1)
        x = x_ref[...]
        neg = jnp.float32(-3.0e38)
        m1 = jnp.max(x, axis=-1)
        eq = x == m1[..., None]
        cnt = jnp.sum(eq.astype(jnp.float32), axis=-1)
        m2 = jnp.max(jnp.where(eq, neg, x), axis=-1)
        m2 = jnp.where(cnt > 1.5, m1, m2)
        gsum = jnp.sum(m1 - m2)

        @pl.when(j == 0)
        def _():
            o_ref[...] = jnp.zeros_like(o_ref)

        o_ref[...] += jnp.full((1, 1, 128), gsum, jnp.float32)

    return pl.pallas_call(
        tc_body,
        grid=(B, nblk),
        in_specs=[pl.BlockSpec((1, bs, V), lambda b, j: (b, s0 // bs + j, 0))],
        out_specs=pl.BlockSpec((1, 1, 128), lambda b, j: (b, 0, 0)),
        out_shape=jax.ShapeDtypeStruct((B, 1, 128), jnp.float32),
    )


def kernel(logits):
    B, S, V = logits.shape
    n_workers = 2 * 16
    x = logits.reshape(B * S, V)
    sc_sum = _make_sc_kernel(V, n_workers, S, SC_SEQ)(x)[:, 0]
    if SC_SEQ < S:
        tc_sum = _make_tc_kernel(B, S, V, SC_SEQ, TC_BS)(logits)[:, 0, 0]
        gap = (sc_sum + tc_sum) * jnp.float32(1.0 / S)
    else:
        gap = sc_sum * jnp.float32(1.0 / S)
    avg_gap = jnp.mean(gap)
    should_exit = avg_gap >= jnp.float32(7.5)
    return gap, avg_gap, should_exit
